# Initial kernel scaffold; baseline (speedup 1.0000x reference)
import functools

import jax
import jax.numpy as jnp
from jax import lax
from jax.experimental import pallas as pl
from jax.experimental.pallas import tpu as pltpu
from jax.experimental.pallas import tpu_sc as plsc

NUM_STEPS = 1000
BATCH = 16384
LANES = 16

_info = plsc.get_sparse_core_info()
_NC, _NS = _info.num_cores, _info.num_subcores
_NW = _NC * _NS
_B_PER_W = BATCH // _NW


@functools.partial(
    pl.kernel,
    mesh=plsc.VectorSubcoreMesh(core_axis_name="c", subcore_axis_name="s"),
    out_type=(
        jax.ShapeDtypeStruct((BATCH,), jnp.float32),
        jax.ShapeDtypeStruct((BATCH,), jnp.float32),
    ),
    scratch_types=[
        pltpu.VMEM((NUM_STEPS,), jnp.float32),
        pltpu.VMEM((NUM_STEPS,), jnp.float32),
        pltpu.VMEM((_B_PER_W,), jnp.int32),
        pltpu.VMEM((_B_PER_W,), jnp.float32),
        pltpu.VMEM((_B_PER_W,), jnp.float32),
    ],
)
def _sc_lookup(beta_hbm, alpha_hbm, t_hbm, beta_out, alpha_out,
               beta_v, alpha_v, idx_v, ob_v, oa_v):
    wid = lax.axis_index("s") * _NC + lax.axis_index("c")
    base = wid * _B_PER_W
    pltpu.sync_copy(beta_hbm, beta_v)
    pltpu.sync_copy(alpha_hbm, alpha_v)
    pltpu.sync_copy(t_hbm.at[pl.ds(base, _B_PER_W)], idx_v)
    for i in range(_B_PER_W // LANES):
        iv = idx_v[pl.ds(i * LANES, LANES)]
        ob_v[pl.ds(i * LANES, LANES)] = plsc.load_gather(beta_v, [iv])
        oa_v[pl.ds(i * LANES, LANES)] = plsc.load_gather(alpha_v, [iv])
    pltpu.sync_copy(ob_v, beta_out.at[pl.ds(base, _B_PER_W)])
    pltpu.sync_copy(oa_v, alpha_out.at[pl.ds(base, _B_PER_W)])


def kernel(beta, alpha, t):
    return _sc_lookup(beta, alpha, t)


# speedup vs baseline: 8.2368x; 8.2368x over previous
import functools

import jax
import jax.numpy as jnp
from jax import lax
from jax.experimental import pallas as pl
from jax.experimental.pallas import tpu as pltpu
from jax.experimental.pallas import tpu_sc as plsc

NUM_STEPS = 1000
BATCH = 16384
LANES = 16

_info = plsc.get_sparse_core_info()
_NC, _NS = _info.num_cores, _info.num_subcores
_NW = _NC * _NS
_B_PER_W = BATCH // _NW


@functools.partial(
    pl.kernel,
    mesh=plsc.VectorSubcoreMesh(core_axis_name="c", subcore_axis_name="s"),
    compiler_params=pltpu.CompilerParams(needs_layout_passes=False),
    out_type=(
        jax.ShapeDtypeStruct((BATCH,), jnp.float32),
        jax.ShapeDtypeStruct((BATCH,), jnp.float32),
    ),
    scratch_types=[
        pltpu.VMEM((NUM_STEPS,), jnp.float32),
        pltpu.VMEM((NUM_STEPS,), jnp.float32),
        pltpu.VMEM((_B_PER_W,), jnp.int32),
        pltpu.VMEM((_B_PER_W,), jnp.float32),
        pltpu.VMEM((_B_PER_W,), jnp.float32),
    ],
)
def _sc_lookup(beta_hbm, alpha_hbm, t_hbm, beta_out, alpha_out,
               beta_v, alpha_v, idx_v, ob_v, oa_v):
    wid = lax.axis_index("s") * _NC + lax.axis_index("c")
    base = wid * _B_PER_W
    pltpu.sync_copy(beta_hbm, beta_v)
    pltpu.sync_copy(alpha_hbm, alpha_v)
    pltpu.sync_copy(t_hbm.at[pl.ds(base, _B_PER_W)], idx_v)
    for i in range(_B_PER_W // LANES):
        iv = idx_v[pl.ds(i * LANES, LANES)]
        ob_v[pl.ds(i * LANES, LANES)] = plsc.load_gather(beta_v, [iv])
        oa_v[pl.ds(i * LANES, LANES)] = plsc.load_gather(alpha_v, [iv])
    pltpu.sync_copy(ob_v, beta_out.at[pl.ds(base, _B_PER_W)])
    pltpu.sync_copy(oa_v, alpha_out.at[pl.ds(base, _B_PER_W)])


def kernel(beta, alpha, t):
    return _sc_lookup(beta, alpha, t)


# speedup vs baseline: 8.3937x; 1.0190x over previous
import functools

import jax
import jax.numpy as jnp
from jax import lax
from jax.experimental import pallas as pl
from jax.experimental.pallas import tpu as pltpu
from jax.experimental.pallas import tpu_sc as plsc

NUM_STEPS = 1000
BATCH = 16384
LANES = 16

_info = plsc.get_sparse_core_info()
_NC, _NS = _info.num_cores, _info.num_subcores
_NW = _NC * _NS
_B_PER_W = BATCH // _NW


@functools.partial(
    pl.kernel,
    mesh=plsc.VectorSubcoreMesh(core_axis_name="c", subcore_axis_name="s"),
    compiler_params=pltpu.CompilerParams(needs_layout_passes=False),
    out_type=(
        jax.ShapeDtypeStruct((BATCH,), jnp.float32),
        jax.ShapeDtypeStruct((BATCH,), jnp.float32),
    ),
    scratch_types=[
        pltpu.VMEM((NUM_STEPS,), jnp.float32),
        pltpu.VMEM((NUM_STEPS,), jnp.float32),
        pltpu.VMEM((_B_PER_W,), jnp.int32),
        pltpu.VMEM((_B_PER_W,), jnp.float32),
        pltpu.VMEM((_B_PER_W,), jnp.float32),
        pltpu.SemaphoreType.DMA,
    ],
)
def _sc_lookup(beta_hbm, alpha_hbm, t_hbm, beta_out, alpha_out,
               beta_v, alpha_v, idx_v, ob_v, oa_v, sem):
    wid = lax.axis_index("s") * _NC + lax.axis_index("c")
    base = wid * _B_PER_W
    c1 = pltpu.async_copy(beta_hbm, beta_v, sem)
    c2 = pltpu.async_copy(alpha_hbm, alpha_v, sem)
    c3 = pltpu.async_copy(t_hbm.at[pl.ds(base, _B_PER_W)], idx_v, sem)
    c1.wait()
    c2.wait()
    c3.wait()
    for i in range(_B_PER_W // LANES):
        iv = idx_v[pl.ds(i * LANES, LANES)]
        ob_v[pl.ds(i * LANES, LANES)] = plsc.load_gather(beta_v, [iv])
        oa_v[pl.ds(i * LANES, LANES)] = plsc.load_gather(alpha_v, [iv])
    c4 = pltpu.async_copy(ob_v, beta_out.at[pl.ds(base, _B_PER_W)], sem)
    c5 = pltpu.async_copy(oa_v, alpha_out.at[pl.ds(base, _B_PER_W)], sem)
    c4.wait()
    c5.wait()


def kernel(beta, alpha, t):
    return _sc_lookup(beta, alpha, t)


# speedup vs baseline: 8.4791x; 1.0102x over previous
import functools

import jax
import jax.numpy as jnp
from jax import lax
from jax.experimental import pallas as pl
from jax.experimental.pallas import tpu as pltpu
from jax.experimental.pallas import tpu_sc as plsc

NUM_STEPS = 1000
BATCH = 16384
LANES = 16

_info = plsc.get_sparse_core_info()


def _build(num_cores):
    nw = num_cores * _info.num_subcores
    b_per_w = BATCH // nw

    @functools.partial(
        pl.kernel,
        mesh=plsc.VectorSubcoreMesh(
            core_axis_name="c", subcore_axis_name="s", num_cores=num_cores
        ),
        compiler_params=pltpu.CompilerParams(needs_layout_passes=False),
        out_type=(
            jax.ShapeDtypeStruct((BATCH,), jnp.float32),
            jax.ShapeDtypeStruct((BATCH,), jnp.float32),
        ),
        scratch_types=[
            pltpu.VMEM((NUM_STEPS,), jnp.float32),
            pltpu.VMEM((NUM_STEPS,), jnp.float32),
            pltpu.VMEM((b_per_w,), jnp.int32),
            pltpu.VMEM((b_per_w,), jnp.float32),
            pltpu.VMEM((b_per_w,), jnp.float32),
            pltpu.SemaphoreType.DMA,
            pltpu.SemaphoreType.DMA,
        ],
    )
    def _sc_lookup(beta_hbm, alpha_hbm, t_hbm, beta_out, alpha_out,
                   beta_v, alpha_v, idx_v, ob_v, oa_v, sem, sem2):
        wid = lax.axis_index("s") * num_cores + lax.axis_index("c")
        base = wid * b_per_w
        c1 = pltpu.async_copy(beta_hbm, beta_v, sem)
        c3 = pltpu.async_copy(t_hbm.at[pl.ds(base, b_per_w)], idx_v, sem)
        c2 = pltpu.async_copy(alpha_hbm, alpha_v, sem2)
        c1.wait()
        c3.wait()
        for i in range(b_per_w // LANES):
            iv = idx_v[pl.ds(i * LANES, LANES)]
            ob_v[pl.ds(i * LANES, LANES)] = plsc.load_gather(beta_v, [iv])
        c4 = pltpu.async_copy(ob_v, beta_out.at[pl.ds(base, b_per_w)], sem)
        c2.wait()
        for i in range(b_per_w // LANES):
            iv = idx_v[pl.ds(i * LANES, LANES)]
            oa_v[pl.ds(i * LANES, LANES)] = plsc.load_gather(alpha_v, [iv])
        c5 = pltpu.async_copy(oa_v, alpha_out.at[pl.ds(base, b_per_w)], sem2)
        c4.wait()
        c5.wait()

    return _sc_lookup


_lookup = _build(_info.num_cores)


def kernel(beta, alpha, t):
    return _lookup(beta, alpha, t)
